# baseline (speedup 1.0000x reference)
import functools

import jax
import jax.numpy as jnp
from jax import lax
from jax.experimental import pallas as pl
from jax.experimental.pallas import tpu as pltpu
from jax.experimental.pallas import tpu_sc as plsc

H = 128
N_PAD = 10240
SPLIT = 5120
EB = 4096
NSTEP = 20
E_PAD = 2 * NSTEP * EB


def _matmul_body(x_ref, ws_ref, wi_ref, wx_ref, ps_ref, pi_ref, px_ref):
    xb = x_ref[...]
    ps_ref[...] = jnp.dot(xb, ws_ref[...], preferred_element_type=jnp.float32)
    pi_ref[...] = jnp.dot(xb, wi_ref[...], preferred_element_type=jnp.float32)
    wxr = wx_ref[:, :H] + wx_ref[:, H:]
    px_ref[...] = jnp.dot(xb, wxr, preferred_element_type=jnp.float32)


def _matmuls(x_pad, W_sum, W_int, W_x):
    bm = 1024
    return pl.pallas_call(
        _matmul_body,
        grid=(N_PAD // bm,),
        in_specs=[
            pl.BlockSpec((bm, H), lambda i: (i, 0)),
            pl.BlockSpec((H, H), lambda i: (0, 0)),
            pl.BlockSpec((H, H), lambda i: (0, 0)),
            pl.BlockSpec((H, 2 * H), lambda i: (0, 0)),
        ],
        out_specs=[pl.BlockSpec((bm, H), lambda i: (i, 0))] * 3,
        out_shape=[jax.ShapeDtypeStruct((N_PAD, H), jnp.float32)] * 3,
    )(x_pad, W_sum, W_int, W_x)


def _ys_body(psum_hbm, dom_hbm, rb_hbm, z_hbm, ys_hbm,
             accum, zbuf, rbuf, ibuf, sbuf, rbv, dbuf, sem):
    c = lax.axis_index("c")
    s = lax.axis_index("s")
    iota = lax.iota(jnp.int32, 16)
    pltpu.sync_copy(z_hbm, zbuf)
    pltpu.sync_copy(rb_hbm.at[pl.ds(pl.multiple_of(8 * c, 8), 16)], rbv)
    base_z = 320 * s
    pltpu.sync_copy(zbuf, accum.at[pl.ds(base_z, 128)])
    pltpu.sync_copy(zbuf, accum.at[pl.ds(base_z + 128, 128)])
    pltpu.sync_copy(zbuf.at[pl.ds(0, 64)], accum.at[pl.ds(base_z + 256, 64)])

    @pl.when(s == 15)
    def _():
        pltpu.sync_copy(zbuf.at[pl.ds(0, 8)], accum.at[pl.ds(5120, 8)])

    plsc.subcore_barrier()

    rvec = rbv[...]
    b0 = rvec[0]
    b1 = rvec[1]
    j0 = b0 + ((b1 - b0) * s >> 4)
    j1 = b0 + ((b1 - b0) * (s + 1) >> 4)
    ja0 = j0 - (j0 & 7)
    trips = (j1 - ja0 + 127) >> 7
    id_base = SPLIT * c

    def ebody(t, carry):
        ja = pl.multiple_of(ja0 + (t << 7), 8)
        pltpu.sync_copy(dom_hbm.at[pl.ds(ja, 128)], ibuf)
        pltpu.sync_copy(psum_hbm.at[pl.ds(ja, 128)], rbuf)
        for v in range(8):
            sl = pl.ds(v * 16, 16)
            jv = ja + v * 16 + iota
            valid = (jv >= j0) & (jv < j1)
            sbuf[sl] = jnp.where(valid, ibuf[sl] - id_base, 5120)
        pltpu.async_copy(rbuf, accum.at[sbuf], sem, add=True).wait()
        return carry

    lax.fori_loop(0, trips, ebody, 0)
    plsc.subcore_barrier()

    for k in range(5):
        pltpu.sync_copy(accum.at[pl.ds(base_z + 64 * k, 64)], dbuf)
        pltpu.sync_copy(dbuf, ys_hbm.at[pl.ds(id_base + base_z + 64 * k, 64)])


def _ys_call(P_sum, dom_pad, rb, zrows):
    mesh = plsc.VectorSubcoreMesh(core_axis_name="c", subcore_axis_name="s")
    f = pl.kernel(
        _ys_body,
        out_type=jax.ShapeDtypeStruct((N_PAD, H), jnp.float32),
        mesh=mesh,
        scratch_types=[
            pltpu.VMEM_SHARED((5128, H), jnp.float32),
            pltpu.VMEM((128, H), jnp.float32),
            pltpu.VMEM((128, H), jnp.float32),
            pltpu.VMEM((128,), jnp.int32),
            pltpu.VMEM((128,), jnp.int32),
            pltpu.VMEM((16,), jnp.int32),
            pltpu.VMEM((64, H), jnp.float32),
            pltpu.SemaphoreType.DMA,
        ],
    )
    return f(P_sum, dom_pad, rb, zrows)


def _main_body(pint_hbm, px_hbm, ys_hbm, ii_hbm, nm_hbm, dm0_hbm, dm1_hbm,
               bnd_hbm, z_hbm, msg_hbm,
               accum, zbuf, rows, iibuf, nmbuf, gibuf, sibuf, oibuf,
               bndv, sem):
    c = lax.axis_index("c")
    s = lax.axis_index("s")
    iota = lax.iota(jnp.int32, 16)
    pltpu.sync_copy(z_hbm, zbuf)

    def step_body(t, carry):
        b = c * NSTEP + t
        e_lo = b << 12
        pltpu.sync_copy(zbuf, accum.at[pl.ds(256 * s, 128)])
        pltpu.sync_copy(zbuf, accum.at[pl.ds(256 * s + 128, 128)])
        plsc.subcore_barrier()
        pltpu.sync_copy(bnd_hbm.at[pl.ds(pl.multiple_of(b << 3, 8), 16)], bndv)
        bvec = bndv[...]
        b0 = bvec[0]
        b1 = bvec[1]
        j0 = b0 + ((b1 - b0) * s >> 4)
        j1 = b0 + ((b1 - b0) * (s + 1) >> 4)
        ja0 = j0 - (j0 & 7)
        trips = (j1 - ja0 + 127) >> 7

        def ebody(u, ecarry):
            ja = pl.multiple_of(ja0 + (u << 7), 8)
            pltpu.sync_copy(ii_hbm.at[pl.ds(ja, 128)], iibuf)
            pltpu.sync_copy(nm_hbm.at[pl.ds(ja, 128)], nmbuf)
            for v in range(8):
                sl = pl.ds(v * 16, 16)
                jv = ja + v * 16 + iota
                valid = (jv >= j0) & (jv < j1)
                gibuf[sl] = jnp.where(valid, nmbuf[sl], 10000)
                sibuf[sl] = jnp.where(valid, iibuf[sl] - e_lo, 0)
            pltpu.async_copy(pint_hbm.at[gibuf], rows, sem).wait()
            pltpu.async_copy(rows, accum.at[sibuf], sem, add=True).wait()
            return ecarry

        lax.fori_loop(0, trips, ebody, 0)
        plsc.subcore_barrier()
        for k in range(2):
            la = 256 * s + 128 * k
            a = pl.multiple_of(e_lo + la, 8)
            for v in range(8):
                oibuf[pl.ds(v * 16, 16)] = la + v * 16 + iota
            pltpu.sync_copy(dm1_hbm.at[pl.ds(a, 128)], gibuf)
            pltpu.async_copy(ys_hbm.at[gibuf], rows, sem).wait()
            pltpu.async_copy(rows, accum.at[oibuf], sem, add=True).wait()
            pltpu.sync_copy(dm0_hbm.at[pl.ds(a, 128)], gibuf)
            pltpu.async_copy(px_hbm.at[gibuf], rows, sem).wait()
            pltpu.async_copy(rows, accum.at[oibuf], sem, add=True).wait()
            pltpu.sync_copy(accum.at[pl.ds(la, 128)], rows)
            pltpu.sync_copy(rows, msg_hbm.at[pl.ds(a, 128)])
        plsc.subcore_barrier()
        return carry

    lax.fori_loop(0, NSTEP, step_body, 0)


def _main_call(P_int, P_x, ys, ii_pad, nm_pad, dm0, dm1, bnd, zrows):
    mesh = plsc.VectorSubcoreMesh(core_axis_name="c", subcore_axis_name="s")
    f = pl.kernel(
        _main_body,
        out_type=jax.ShapeDtypeStruct((E_PAD, H), jnp.float32),
        mesh=mesh,
        scratch_types=[
            pltpu.VMEM_SHARED((EB, H), jnp.float32),
            pltpu.VMEM((128, H), jnp.float32),
            pltpu.VMEM((128, H), jnp.float32),
            pltpu.VMEM((128,), jnp.int32),
            pltpu.VMEM((128,), jnp.int32),
            pltpu.VMEM((128,), jnp.int32),
            pltpu.VMEM((128,), jnp.int32),
            pltpu.VMEM((128,), jnp.int32),
            pltpu.VMEM((16,), jnp.int32),
            pltpu.SemaphoreType.DMA,
        ],
    )
    return f(P_int, P_x, ys, ii_pad, nm_pad, dm0, dm1, bnd, zrows)


def _bn_body(ed_f, msg_ref, g_ref, bta_ref, out_ref, s_acc, q_acc):
    p = pl.program_id(0)
    i = pl.program_id(1)

    @pl.when(jnp.logical_and(p == 0, i == 0))
    def _():
        s_acc[...] = jnp.zeros_like(s_acc)
        q_acc[...] = jnp.zeros_like(q_acc)

    blk = msg_ref[...]

    @pl.when(p == 0)
    def _():
        s_acc[...] += jnp.sum(blk, axis=0, keepdims=True)
        q_acc[...] += jnp.sum(blk * blk, axis=0, keepdims=True)
        out_ref[...] = blk

    @pl.when(p == 1)
    def _():
        mean = s_acc[...] / ed_f
        var = q_acc[...] / ed_f - mean * mean
        inv = lax.rsqrt(var + 1e-5) * g_ref[...]
        out_ref[...] = jnp.maximum((blk - mean) * inv + bta_ref[...], 0.0)


def _bn_call(msg, gamma, beta, ed):
    bm = 2048
    return pl.pallas_call(
        functools.partial(_bn_body, float(ed)),
        grid=(2, E_PAD // bm),
        in_specs=[
            pl.BlockSpec((bm, H), lambda p, i: (i, 0)),
            pl.BlockSpec((1, H), lambda p, i: (0, 0)),
            pl.BlockSpec((1, H), lambda p, i: (0, 0)),
        ],
        out_specs=pl.BlockSpec((bm, H), lambda p, i: (i, 0)),
        out_shape=jax.ShapeDtypeStruct((E_PAD, H), jnp.float32),
        scratch_shapes=[
            pltpu.VMEM((1, H), jnp.float32),
            pltpu.VMEM((1, H), jnp.float32),
        ],
    )(msg, gamma, beta)


def kernel(x, y, domain_indicator, node_map_edge_index, intersect_indicator,
           domain_map_edge_index, W_sum, W_int, W_x, bn_gamma, bn_beta):
    n = x.shape[0]
    en = node_map_edge_index.shape[1]
    ed = domain_map_edge_index.shape[1]

    x_pad = jnp.zeros((N_PAD, H), jnp.float32).at[:n].set(x)
    P_sum, P_int, P_x = _matmuls(x_pad, W_sum, W_int, W_x)

    dom = domain_indicator.astype(jnp.int32)
    dom_pad = jnp.zeros((N_PAD,), jnp.int32).at[:n].set(dom)
    rb1 = jnp.searchsorted(dom, SPLIT).astype(jnp.int32)
    rb = (jnp.zeros((24,), jnp.int32)
          .at[1].set(rb1).at[8].set(rb1).at[9].set(n))
    zrows = jnp.zeros((128, H), jnp.float32)
    ys = _ys_call(P_sum, dom_pad, rb, zrows)

    ii = intersect_indicator.astype(jnp.int32)
    ii_pad = jnp.zeros((en + 128,), jnp.int32).at[:en].set(ii)
    nm_pad = jnp.zeros((en + 128,), jnp.int32).at[:en].set(
        node_map_edge_index[1].astype(jnp.int32))
    nblocks = E_PAD // EB
    bounds = jnp.searchsorted(
        ii, jnp.arange(nblocks + 1, dtype=jnp.int32) * EB).astype(jnp.int32)
    ar = jnp.arange(nblocks) * 8
    bnd = (jnp.zeros((8 * nblocks + 16,), jnp.int32)
           .at[ar].set(bounds[:nblocks])
           .at[ar + 1].set(bounds[1:nblocks + 1]))
    dm0 = jnp.full((E_PAD,), n, jnp.int32).at[:ed].set(
        domain_map_edge_index[0].astype(jnp.int32))
    dm1 = jnp.full((E_PAD,), n, jnp.int32).at[:ed].set(
        domain_map_edge_index[1].astype(jnp.int32))

    msg = _main_call(P_int, P_x, ys, ii_pad, nm_pad, dm0, dm1, bnd, zrows)
    out = _bn_call(msg, bn_gamma.reshape(1, H), bn_beta.reshape(1, H), ed)
    return out[:ed]
